# baseline (speedup 1.0000x reference)
import functools

import jax
import jax.numpy as jnp
from jax.experimental import pallas as pl
from jax.experimental.pallas import tpu as pltpu

E = 8
K = 2
D = 1024
C = 1024
B = 2
N = 2048
T = B * N
TT = 512
NT = T // TT
TILES_PER_B = NT // B

_F32 = jnp.float32
_HI = jax.lax.Precision.HIGHEST


def _erf(x):
    s = jnp.sign(x)
    a = jnp.abs(x)
    t = 1.0 / (1.0 + 0.3275911 * a)
    poly = t * (0.254829592
                + t * (-0.284496736
                       + t * (1.421413741
                              + t * (-1.453152027 + t * 1.061405429))))
    return s * (1.0 - poly * jnp.exp(-a * a))


def _gelu_exact(x):
    return 0.5 * x * (1.0 + _erf(x * 0.7071067811865476))


def _gate_kernel(x_ref, gw_ref, combine_ref, bal_ref, z_ref,
                 proxy_acc, dens_acc, z_acc):
    i = pl.program_id(0)
    x = x_ref[...]
    gw = gw_ref[...]
    logits = jax.lax.dot_general(x, gw, (((1,), (1,)), ((), ())),
                                 preferred_element_type=_F32)
    m = jnp.max(logits, axis=1, keepdims=True)
    ex = jnp.exp(logits - m)
    sumex = jnp.sum(ex, axis=1, keepdims=True)
    lse = m[:, 0] + jnp.log(sumex[:, 0])
    p = ex / sumex

    lane = jax.lax.broadcasted_iota(jnp.int32, p.shape, 1)
    m1 = jnp.max(p, axis=1, keepdims=True)
    i1 = jnp.min(jnp.where(p == m1, lane, E), axis=1, keepdims=True)
    oh1 = lane == i1
    p2 = jnp.where(oh1, -jnp.inf, p)
    m2 = jnp.max(p2, axis=1, keepdims=True)
    i2 = jnp.min(jnp.where(p2 == m2, lane, E), axis=1, keepdims=True)
    oh2 = lane == i2
    denom = m1 + m2
    combine = (oh1.astype(_F32) * (m1 / denom)
               + oh2.astype(_F32) * (m2 / denom))
    combine_ref[...] = combine

    @pl.when(i == 0)
    def _():
        proxy_acc[...] = jnp.zeros_like(proxy_acc)
        dens_acc[...] = jnp.zeros_like(dens_acc)
        z_acc[...] = jnp.zeros_like(z_acc)

    b = i // TILES_PER_B
    rows = jax.lax.broadcasted_iota(jnp.int32, (B, E), 0)
    sel = (rows == b).astype(_F32)
    proxy_acc[...] += sel * jnp.sum(p, axis=0)[None, :]
    dens_acc[...] += sel * jnp.sum(oh1.astype(_F32), axis=0)[None, :]
    z_acc[...] += jnp.sum(lse * lse).reshape(1, 1)

    @pl.when(i == NT - 1)
    def _():
        proxy = proxy_acc[...] / jnp.float32(N)
        dens = dens_acc[...] / jnp.float32(N)
        bal_ref[...] = (jnp.sum(proxy * dens) / jnp.float32(B * E)
                        * jnp.float32(E * E)).reshape(1, 1)
        z_ref[...] = (z_acc[0, 0] / jnp.float32(T)).reshape(1, 1)


def _gate(x2d, gate_w):
    return pl.pallas_call(
        _gate_kernel,
        grid=(NT,),
        in_specs=[
            pl.BlockSpec((TT, D), lambda i: (i, 0)),
            pl.BlockSpec((E, D), lambda i: (0, 0)),
        ],
        out_specs=[
            pl.BlockSpec((TT, E), lambda i: (i, 0)),
            pl.BlockSpec((1, 1), lambda i: (0, 0)),
            pl.BlockSpec((1, 1), lambda i: (0, 0)),
        ],
        out_shape=[
            jax.ShapeDtypeStruct((T, E), _F32),
            jax.ShapeDtypeStruct((1, 1), _F32),
            jax.ShapeDtypeStruct((1, 1), _F32),
        ],
        scratch_shapes=[
            pltpu.VMEM((B, E), _F32),
            pltpu.VMEM((B, E), _F32),
            pltpu.VMEM((1, 1), _F32),
        ],
    )(x2d, gate_w)


def _moe_kernel(x_ref, w1_ref, b1_ref, w2_ref, b2_ref, combine_ref, out_ref):
    e = pl.program_id(1)
    lane = jax.lax.broadcasted_iota(jnp.int32, (TT, E), 1)
    col = jnp.sum(jnp.where(lane == e, combine_ref[...], 0.0), axis=1)

    h = jax.lax.dot_general(x_ref[...], w1_ref[0], (((1,), (1,)), ((), ())),
                            precision=_HI, preferred_element_type=_F32)
    h = h + b1_ref[0]
    h = _gelu_exact(h)
    o = jax.lax.dot_general(h, w2_ref[0], (((1,), (1,)), ((), ())),
                            precision=_HI, preferred_element_type=_F32)
    o = o + b2_ref[0]
    contrib = col[:, None] * o

    @pl.when(e == 0)
    def _():
        out_ref[...] = contrib

    @pl.when(e > 0)
    def _():
        out_ref[...] += contrib


def _moe(x2d, w1, b1, w2, b2, combine):
    return pl.pallas_call(
        _moe_kernel,
        grid=(NT, E),
        in_specs=[
            pl.BlockSpec((TT, D), lambda t, e: (t, 0)),
            pl.BlockSpec((1, C, D), lambda t, e: (e, 0, 0)),
            pl.BlockSpec((1, 1, C), lambda t, e: (e, 0, 0)),
            pl.BlockSpec((1, C, C), lambda t, e: (e, 0, 0)),
            pl.BlockSpec((1, 1, C), lambda t, e: (e, 0, 0)),
            pl.BlockSpec((TT, E), lambda t, e: (t, 0)),
        ],
        out_specs=pl.BlockSpec((TT, C), lambda t, e: (t, 0)),
        out_shape=jax.ShapeDtypeStruct((T, C), _F32),
    )(x2d, w1, b1.reshape(E, 1, C), w2, b2.reshape(E, 1, C), combine)


@jax.jit
def kernel(x_img, gate_w, w1, b1, w2, b2):
    x2d = x_img.reshape(T, D)
    combine, bal, z = _gate(x2d, gate_w)
    out = _moe(x2d, w1, b1, w2, b2, combine)
    return (out.reshape(B, N, C), bal[0, 0], z[0, 0])


# speedup vs baseline: 3.2209x; 3.2209x over previous
import functools

import jax
import jax.numpy as jnp
from jax.experimental import pallas as pl
from jax.experimental.pallas import tpu as pltpu

E = 8
K = 2
D = 1024
C = 1024
B = 2
N = 2048
T = B * N
TT = 512
NT = T // TT
TILES_PER_B = NT // B

_F32 = jnp.float32
_HI = jax.lax.Precision.HIGHEST


def _erf(x):
    s = jnp.sign(x)
    a = jnp.abs(x)
    t = 1.0 / (1.0 + 0.3275911 * a)
    poly = t * (0.254829592
                + t * (-0.284496736
                       + t * (1.421413741
                              + t * (-1.453152027 + t * 1.061405429))))
    return s * (1.0 - poly * jnp.exp(-a * a))


def _gelu_exact(x):
    return 0.5 * x * (1.0 + _erf(x * 0.7071067811865476))


def _gate_kernel(x_ref, gw_ref, combine_ref, bal_ref, z_ref,
                 proxy_acc, dens_acc, z_acc):
    i = pl.program_id(0)
    x = x_ref[...]
    gw = gw_ref[...]
    logits = jax.lax.dot_general(x, gw, (((1,), (1,)), ((), ())),
                                 preferred_element_type=_F32)
    m = jnp.max(logits, axis=1, keepdims=True)
    ex = jnp.exp(logits - m)
    sumex = jnp.sum(ex, axis=1, keepdims=True)
    lse = m[:, 0] + jnp.log(sumex[:, 0])
    p = ex / sumex

    lane = jax.lax.broadcasted_iota(jnp.int32, p.shape, 1)
    m1 = jnp.max(p, axis=1, keepdims=True)
    i1 = jnp.min(jnp.where(p == m1, lane, E), axis=1, keepdims=True)
    oh1 = lane == i1
    p2 = jnp.where(oh1, -jnp.inf, p)
    m2 = jnp.max(p2, axis=1, keepdims=True)
    i2 = jnp.min(jnp.where(p2 == m2, lane, E), axis=1, keepdims=True)
    oh2 = lane == i2
    denom = m1 + m2
    combine = (oh1.astype(_F32) * (m1 / denom)
               + oh2.astype(_F32) * (m2 / denom))
    combine_ref[...] = combine

    @pl.when(i == 0)
    def _():
        proxy_acc[...] = jnp.zeros_like(proxy_acc)
        dens_acc[...] = jnp.zeros_like(dens_acc)
        z_acc[...] = jnp.zeros_like(z_acc)

    b = i // TILES_PER_B
    rows = jax.lax.broadcasted_iota(jnp.int32, (B, E), 0)
    sel = (rows == b).astype(_F32)
    proxy_acc[...] += sel * jnp.sum(p, axis=0)[None, :]
    dens_acc[...] += sel * jnp.sum(oh1.astype(_F32), axis=0)[None, :]
    z_acc[...] += jnp.sum(lse * lse).reshape(1, 1)

    @pl.when(i == NT - 1)
    def _():
        proxy = proxy_acc[...] / jnp.float32(N)
        dens = dens_acc[...] / jnp.float32(N)
        bal_ref[...] = (jnp.sum(proxy * dens) / jnp.float32(B * E)
                        * jnp.float32(E * E)).reshape(1, 1)
        z_ref[...] = (z_acc[0, 0] / jnp.float32(T)).reshape(1, 1)


def _gate(x2d, gate_w):
    return pl.pallas_call(
        _gate_kernel,
        grid=(NT,),
        in_specs=[
            pl.BlockSpec((TT, D), lambda i: (i, 0)),
            pl.BlockSpec((E, D), lambda i: (0, 0)),
        ],
        out_specs=[
            pl.BlockSpec((TT, E), lambda i: (i, 0)),
            pl.BlockSpec((1, 1), lambda i: (0, 0)),
            pl.BlockSpec((1, 1), lambda i: (0, 0)),
        ],
        out_shape=[
            jax.ShapeDtypeStruct((T, E), _F32),
            jax.ShapeDtypeStruct((1, 1), _F32),
            jax.ShapeDtypeStruct((1, 1), _F32),
        ],
        scratch_shapes=[
            pltpu.VMEM((B, E), _F32),
            pltpu.VMEM((B, E), _F32),
            pltpu.VMEM((1, 1), _F32),
        ],
    )(x2d, gate_w)


def _moe_kernel(x_ref, w1_ref, b1_ref, w2_ref, b2_ref, combine_ref, out_ref):
    e = pl.program_id(1)
    lane = jax.lax.broadcasted_iota(jnp.int32, (TT, E), 1)
    col = jnp.sum(jnp.where(lane == e, combine_ref[...], 0.0), axis=1)

    h = jax.lax.dot_general(x_ref[...], w1_ref[0], (((1,), (1,)), ((), ())),
                            preferred_element_type=_F32)
    h = h + b1_ref[0]
    h = _gelu_exact(h)
    o = jax.lax.dot_general(h, w2_ref[0], (((1,), (1,)), ((), ())),
                            preferred_element_type=_F32)
    o = o + b2_ref[0]
    contrib = col[:, None] * o

    @pl.when(e == 0)
    def _():
        out_ref[...] = contrib

    @pl.when(e > 0)
    def _():
        out_ref[...] += contrib


def _moe(x2d, w1, b1, w2, b2, combine):
    return pl.pallas_call(
        _moe_kernel,
        grid=(NT, E),
        in_specs=[
            pl.BlockSpec((TT, D), lambda t, e: (t, 0)),
            pl.BlockSpec((1, C, D), lambda t, e: (e, 0, 0)),
            pl.BlockSpec((1, 1, C), lambda t, e: (e, 0, 0)),
            pl.BlockSpec((1, C, C), lambda t, e: (e, 0, 0)),
            pl.BlockSpec((1, 1, C), lambda t, e: (e, 0, 0)),
            pl.BlockSpec((TT, E), lambda t, e: (t, 0)),
        ],
        out_specs=pl.BlockSpec((TT, C), lambda t, e: (t, 0)),
        out_shape=jax.ShapeDtypeStruct((T, C), _F32),
    )(x2d, w1, b1.reshape(E, 1, C), w2, b2.reshape(E, 1, C), combine)


@jax.jit
def kernel(x_img, gate_w, w1, b1, w2, b2):
    x2d = x_img.reshape(T, D)
    combine, bal, z = _gate(x2d, gate_w)
    out = _moe(x2d, w1, b1, w2, b2, combine)
    return (out.reshape(B, N, C), bal[0, 0], z[0, 0])
